# baseline (speedup 1.0000x reference)
import functools

import jax
import jax.numpy as jnp
from jax import lax
from jax.experimental import pallas as pl
from jax.experimental.pallas import tpu as pltpu
from jax.experimental.pallas import tpu_sc as plsc

_B = 16384
_D = 64
_NW = 32
_ROWS = _B // _NW
_CHUNK = 128
_NCH = _ROWS // _CHUNK
_LANES = 16
_CPD = _D // _LANES


def _make_sc_kernel():
    mesh = plsc.VectorSubcoreMesh(core_axis_name="c", subcore_axis_name="s")

    @functools.partial(
        pl.kernel,
        mesh=mesh,
        out_type=jax.ShapeDtypeStruct((_NW * _LANES,), jnp.float32),
        scratch_types=[
            pltpu.VMEM((_NCH, _CHUNK), jnp.int32),
            pltpu.VMEM((_ROWS, _D), jnp.float32),
            pltpu.VMEM((_ROWS, _D), jnp.float32),
            pltpu.VMEM((_LANES,), jnp.float32),
            pltpu.SemaphoreType.DMA,
            pltpu.SemaphoreType.DMA,
        ],
        compiler_params=pltpu.CompilerParams(use_tc_tiling_on_sc=False),
    )
    def sc_kernel(centers_hbm, labels_hbm, feats_hbm, out_hbm,
                  idx_v, rows_v, feats_v, acc_v, sem_g, sem_f):
        wid = lax.axis_index("s") * 2 + lax.axis_index("c")
        base = wid * _ROWS

        pltpu.sync_copy(labels_hbm.at[wid], idx_v)
        feats_cp = pltpu.async_copy(
            feats_hbm.at[pl.ds(base, _ROWS)], feats_v, sem_f)
        gathers = []
        for j in range(_NCH):
            gathers.append(pltpu.async_copy(
                centers_hbm.at[idx_v.at[j]],
                rows_v.at[pl.ds(j * _CHUNK, _CHUNK)],
                sem_g))
        for g in gathers:
            g.wait()
        feats_cp.wait()

        def body(i, accs):
            out = []
            for c in range(_CPD):
                f = feats_v[i, pl.ds(c * _LANES, _LANES)]
                r = rows_v[i, pl.ds(c * _LANES, _LANES)]
                d = f - r
                out.append(accs[c] + d * d)
            return tuple(out)

        zero = jnp.zeros((_LANES,), jnp.float32)
        accs = lax.fori_loop(0, _ROWS, body, (zero,) * _CPD)
        acc_v[...] = (accs[0] + accs[1]) + (accs[2] + accs[3])
        pltpu.sync_copy(acc_v, out_hbm.at[pl.ds(wid * _LANES, _LANES)])

    return sc_kernel


_SC_KERNEL = None


def kernel(feats, labels, centers):
    global _SC_KERNEL
    if _SC_KERNEL is None:
        _SC_KERNEL = _make_sc_kernel()
    labels32 = labels.astype(jnp.int32).reshape(_NW, _NCH, _CHUNK)
    partials = _SC_KERNEL(centers, labels32, feats)
    return jnp.sum(partials) * (0.5 / _B)


# speedup vs baseline: 1.6802x; 1.6802x over previous
import functools

import jax
import jax.numpy as jnp
from jax import lax
from jax.experimental import pallas as pl
from jax.experimental.pallas import tpu as pltpu
from jax.experimental.pallas import tpu_sc as plsc

_B = 16384
_D = 64
_NW = 32
_ROWS = _B // _NW
_CHUNK = 128
_NCH = _ROWS // _CHUNK
_LANES = 16
_CPD = _D // _LANES


def _make_sc_kernel():
    mesh = plsc.VectorSubcoreMesh(core_axis_name="c", subcore_axis_name="s")

    @functools.partial(
        pl.kernel,
        mesh=mesh,
        out_type=jax.ShapeDtypeStruct((_NW * _LANES,), jnp.float32),
        scratch_types=[
            pltpu.VMEM((2, _CHUNK, _D), jnp.float32),
            pltpu.VMEM((2, _CHUNK, _D), jnp.float32),
            pltpu.VMEM((_LANES,), jnp.float32),
            pltpu.VMEM((_ROWS,), jnp.int32),
            pltpu.SemaphoreType.DMA,
            pltpu.SemaphoreType.DMA,
            pltpu.SemaphoreType.DMA,
            pltpu.SemaphoreType.DMA,
        ],
    )
    def sc_kernel(table_hbm, labels_hbm, feats_hbm, out_hbm,
                  rows_v, feats_v, acc_v, lab_v,
                  sem_g0, sem_g1, sem_f0, sem_f1):
        wid = lax.axis_index("s") * 2 + lax.axis_index("c")
        base = wid * _ROWS
        sems_g = (sem_g0, sem_g1)
        sems_f = (sem_f0, sem_f1)

        pltpu.sync_copy(labels_hbm.at[pl.ds(base, _ROWS)], lab_v)

        def fire_chunk(ch):
            p = ch % 2
            fcp = pltpu.async_copy(
                feats_hbm.at[pl.ds(base + ch * _CHUNK, _CHUNK)],
                feats_v.at[p], sems_f[p])

            def fire(g, _, ch=ch, p=p):
                labs = lab_v[pl.ds(ch * _CHUNK + g * _LANES, _LANES)]
                for l in range(_LANES):
                    pltpu.make_async_copy(
                        table_hbm.at[pl.ds(labs[l], 1)],
                        rows_v.at[p].at[pl.ds(g * _LANES + l, 1)],
                        sems_g[p],
                    ).start()
                return 0

            lax.fori_loop(0, _CHUNK // _LANES, fire, 0)
            return fcp

        def drain_chunk(ch, fcp):
            p = ch % 2
            pltpu.make_async_copy(
                feats_hbm.at[pl.ds(0, _CHUNK)], rows_v.at[p], sems_g[p]
            ).wait()
            fcp.wait()

        zero = jnp.zeros((_LANES,), jnp.float32)
        accs = (zero,) * _CPD
        pending = fire_chunk(0)
        for ch in range(_NCH):
            drain_chunk(ch, pending)
            if ch + 1 < _NCH:
                pending = fire_chunk(ch + 1)
            p = ch % 2

            def body(i, a, p=p):
                out = []
                for c in range(_CPD):
                    f = feats_v[p, i, pl.ds(c * _LANES, _LANES)]
                    r = rows_v[p, i, pl.ds(c * _LANES, _LANES)]
                    d = f - r
                    out.append(a[c] + d * d)
                return tuple(out)

            accs = lax.fori_loop(0, _CHUNK, body, accs)

        acc_v[...] = (accs[0] + accs[1]) + (accs[2] + accs[3])
        pltpu.sync_copy(acc_v, out_hbm.at[pl.ds(wid * _LANES, _LANES)])

    return sc_kernel


_SC_KERNEL = None


def kernel(feats, labels, centers):
    global _SC_KERNEL
    if _SC_KERNEL is None:
        _SC_KERNEL = _make_sc_kernel()
    labels32 = labels.astype(jnp.int32)
    partials = _SC_KERNEL(centers, labels32, feats)
    return jnp.sum(partials) * (0.5 / _B)
